# baseline (speedup 1.0000x reference)
import functools

import jax
import jax.numpy as jnp
from jax import lax
from jax.experimental import pallas as pl
from jax.experimental.pallas import tpu as pltpu
from jax.experimental.pallas import tpu_sc as plsc

NUM_ROWS = 4096 * 26
NUM_COLS = 1000
NC = 2
NS = 16
NW = NC * NS
ROWS_PER_W = NUM_ROWS // NW
LANES = 16
ZROWS = 16
ZWORDS = ZROWS * NUM_COLS
NFILL = ROWS_PER_W // ZROWS
IDXW = 128
NIDX = ROWS_PER_W // IDXW

_mesh = plsc.VectorSubcoreMesh(core_axis_name="c", subcore_axis_name="s")


@functools.partial(
    pl.kernel,
    out_type=jax.ShapeDtypeStruct((NUM_ROWS * NUM_COLS,), jnp.float32),
    mesh=_mesh,
    scratch_types=(
        pltpu.VMEM((ROWS_PER_W,), jnp.int32),
        pltpu.VMEM((NIDX, IDXW), jnp.int32),
        pltpu.VMEM((ZWORDS,), jnp.float32),
        pltpu.VMEM((IDXW,), jnp.float32),
        pltpu.VMEM_SHARED((NS, ZWORDS), jnp.float32),
        pltpu.SemaphoreType.DMA,
        pltpu.SemaphoreType.DMA,
    ),
    compiler_params=pltpu.CompilerParams(needs_layout_passes=False),
)
def _one_hot_sc(x_hbm, out_hbm, idx_v, pos_v, zbuf, ones_v, zshared,
                fill_sem, ones_sem):
    sid = lax.axis_index("s")
    wid = sid * NC + lax.axis_index("c")
    base_row = wid * ROWS_PER_W

    pltpu.sync_copy(x_hbm.at[pl.ds(base_row, ROWS_PER_W)], idx_v)

    zeros16 = jnp.zeros((LANES,), jnp.float32)
    ones16 = jnp.ones((LANES,), jnp.float32)
    iota16 = lax.iota(jnp.int32, LANES)

    def _zero(i, carry):
        zbuf[pl.ds(i * LANES, LANES)] = zeros16
        return carry

    lax.fori_loop(0, ZWORDS // LANES, _zero, 0)
    pltpu.sync_copy(zbuf, zshared.at[sid])
    for k in range(IDXW // LANES):
        ones_v[pl.ds(k * LANES, LANES)] = ones16

    zsrc = zshared.at[sid]

    def _fill(c, carry):
        dst = out_hbm.at[pl.ds((base_row + c * ZROWS) * NUM_COLS, ZWORDS)]
        pltpu.make_async_copy(zsrc, dst, fill_sem).start()
        return carry

    lax.fori_loop(0, NFILL, _fill, 0)

    def _pos(r, carry):
        for k in range(IDXW // LANES):
            off = r * IDXW + k * LANES
            idx = idx_v[pl.ds(off, LANES)]
            pos_v[r, pl.ds(k * LANES, LANES)] = (
                (base_row + off + iota16) * NUM_COLS + idx)
        return carry

    lax.fori_loop(0, NIDX, _pos, 0)

    def _drain(c, carry):
        dst = out_hbm.at[pl.ds(base_row * NUM_COLS, ZWORDS)]
        pltpu.make_async_copy(zsrc, dst, fill_sem).wait()
        return carry

    lax.fori_loop(0, NFILL, _drain, 0)

    def _ones(r, carry):
        pltpu.make_async_copy(
            ones_v, out_hbm.at[pos_v.at[r]], ones_sem).start()
        return carry

    lax.fori_loop(0, NIDX, _ones, 0)

    def _odrain(r, carry):
        pltpu.make_async_copy(
            ones_v, out_hbm.at[pos_v.at[0]], ones_sem).wait()
        return carry

    lax.fori_loop(0, NIDX, _odrain, 0)


def kernel(x):
    x = x.reshape(-1).astype(jnp.int32)
    out = _one_hot_sc(x)
    return out.reshape(4096, 26, NUM_COLS)


# speedup vs baseline: 1.0554x; 1.0554x over previous
import functools

import jax
import jax.numpy as jnp
from jax import lax
from jax.experimental import pallas as pl
from jax.experimental.pallas import tpu as pltpu
from jax.experimental.pallas import tpu_sc as plsc

NUM_ROWS = 4096 * 26
NUM_COLS = 1000
NC = 2
NS = 16
NW = NC * NS
ROWS_PER_W = NUM_ROWS // NW
LANES = 16
ZROWS = 16
ZWORDS = ZROWS * NUM_COLS
NFILL = ROWS_PER_W // ZROWS
IDXW = 128
NIDX = ROWS_PER_W // IDXW

_mesh = plsc.VectorSubcoreMesh(core_axis_name="c", subcore_axis_name="s")


@functools.partial(
    pl.kernel,
    out_type=jax.ShapeDtypeStruct((NUM_ROWS * NUM_COLS,), jnp.float32),
    mesh=_mesh,
    scratch_types=(
        pltpu.VMEM((ROWS_PER_W,), jnp.int32),
        pltpu.VMEM((NIDX, IDXW), jnp.int32),
        pltpu.VMEM((ZWORDS,), jnp.float32),
        pltpu.VMEM((IDXW,), jnp.float32),
        pltpu.SemaphoreType.DMA,
        pltpu.SemaphoreType.DMA,
    ),
    compiler_params=pltpu.CompilerParams(needs_layout_passes=False),
)
def _one_hot_sc(x_hbm, out_hbm, idx_v, pos_v, zbuf, ones_v,
                fill_sem, ones_sem):
    wid = lax.axis_index("s") * NC + lax.axis_index("c")
    base_row = wid * ROWS_PER_W

    pltpu.sync_copy(x_hbm.at[pl.ds(base_row, ROWS_PER_W)], idx_v)

    zeros16 = jnp.zeros((LANES,), jnp.float32)
    ones16 = jnp.ones((LANES,), jnp.float32)
    iota16 = lax.iota(jnp.int32, LANES)

    def _zero(i, carry):
        zbuf[pl.ds(i * LANES, LANES)] = zeros16
        return carry

    lax.fori_loop(0, ZWORDS // LANES, _zero, 0)
    for k in range(IDXW // LANES):
        ones_v[pl.ds(k * LANES, LANES)] = ones16

    zsrc = zbuf

    def _fill(c, carry):
        dst = out_hbm.at[pl.ds((base_row + c * ZROWS) * NUM_COLS, ZWORDS)]
        pltpu.make_async_copy(zsrc, dst, fill_sem).start()
        return carry

    lax.fori_loop(0, NFILL, _fill, 0)

    def _pos(r, carry):
        for k in range(IDXW // LANES):
            off = r * IDXW + k * LANES
            idx = idx_v[pl.ds(off, LANES)]
            pos_v[r, pl.ds(k * LANES, LANES)] = (
                (base_row + off + iota16) * NUM_COLS + idx)
        return carry

    lax.fori_loop(0, NIDX, _pos, 0)

    def _drain(c, carry):
        dst = out_hbm.at[pl.ds(base_row * NUM_COLS, ZWORDS)]
        pltpu.make_async_copy(zsrc, dst, fill_sem).wait()
        return carry

    lax.fori_loop(0, NFILL, _drain, 0)

    def _ones(r, carry):
        pltpu.make_async_copy(
            ones_v, out_hbm.at[pos_v.at[r]], ones_sem).start()
        return carry

    lax.fori_loop(0, NIDX, _ones, 0)

    def _odrain(r, carry):
        pltpu.make_async_copy(
            ones_v, out_hbm.at[pos_v.at[0]], ones_sem).wait()
        return carry

    lax.fori_loop(0, NIDX, _odrain, 0)


def kernel(x):
    x = x.reshape(-1).astype(jnp.int32)
    out = _one_hot_sc(x)
    return out.reshape(4096, 26, NUM_COLS)


# speedup vs baseline: 1.0556x; 1.0001x over previous
import functools

import jax
import jax.numpy as jnp
from jax import lax
from jax.experimental import pallas as pl
from jax.experimental.pallas import tpu as pltpu
from jax.experimental.pallas import tpu_sc as plsc

NUM_ROWS = 4096 * 26
NUM_COLS = 1000
NC = 2
NS = 16
NW = NC * NS
ROWS_PER_W = NUM_ROWS // NW
LANES = 16
ZROWS = 16
ZWORDS = ZROWS * NUM_COLS
NFILL = ROWS_PER_W // ZROWS
NPAIR = NFILL // 2
DP = 4
IDXW = 128
NIDX = ROWS_PER_W // IDXW

_mesh = plsc.VectorSubcoreMesh(core_axis_name="c", subcore_axis_name="s")


@functools.partial(
    pl.kernel,
    out_type=jax.ShapeDtypeStruct((NUM_ROWS * NUM_COLS,), jnp.float32),
    mesh=_mesh,
    scratch_types=(
        pltpu.VMEM((ROWS_PER_W,), jnp.int32),
        pltpu.VMEM((NIDX, IDXW), jnp.int32),
        pltpu.VMEM((ZWORDS,), jnp.float32),
        pltpu.VMEM((IDXW,), jnp.float32),
        pltpu.VMEM_SHARED((NS, ZWORDS), jnp.float32),
        pltpu.SemaphoreType.DMA,
        pltpu.SemaphoreType.DMA,
        pltpu.SemaphoreType.DMA,
    ),
    compiler_params=pltpu.CompilerParams(needs_layout_passes=False),
)
def _one_hot_sc(x_hbm, out_hbm, idx_v, pos_v, zbuf, ones_v, zshared,
                sem_a, sem_b, ones_sem):
    sid = lax.axis_index("s")
    wid = sid * NC + lax.axis_index("c")
    base_row = wid * ROWS_PER_W

    pltpu.sync_copy(x_hbm.at[pl.ds(base_row, ROWS_PER_W)], idx_v)

    zeros16 = jnp.zeros((LANES,), jnp.float32)
    ones16 = jnp.ones((LANES,), jnp.float32)
    iota16 = lax.iota(jnp.int32, LANES)

    def _zero(i, carry):
        zbuf[pl.ds(i * LANES, LANES)] = zeros16
        return carry

    lax.fori_loop(0, ZWORDS // LANES, _zero, 0)
    pltpu.sync_copy(zbuf, zshared.at[sid])
    for k in range(IDXW // LANES):
        ones_v[pl.ds(k * LANES, LANES)] = ones16

    zsh = zshared.at[sid]

    def _dsta(g):
        return out_hbm.at[pl.ds((base_row + 2 * g * ZROWS) * NUM_COLS,
                                ZWORDS)]

    def _dstb(g):
        return out_hbm.at[pl.ds((base_row + (2 * g + 1) * ZROWS) * NUM_COLS,
                                ZWORDS)]

    def _fire(g):
        pltpu.make_async_copy(zbuf, _dsta(g), sem_a).start()
        pltpu.make_async_copy(zsh, _dstb(g), sem_b).start()

    def _wait_pair():
        pltpu.make_async_copy(zbuf, _dsta(0), sem_a).wait()
        pltpu.make_async_copy(zsh, _dstb(0), sem_b).wait()

    def _prime(g, carry):
        _fire(g)
        return carry

    lax.fori_loop(0, DP, _prime, 0)

    def _steady(g, carry):
        _fire(g)
        _wait_pair()
        return carry

    lax.fori_loop(DP, NPAIR, _steady, 0)

    def _pos(r, carry):
        for k in range(IDXW // LANES):
            off = r * IDXW + k * LANES
            idx = idx_v[pl.ds(off, LANES)]
            pos_v[r, pl.ds(k * LANES, LANES)] = (
                (base_row + off + iota16) * NUM_COLS + idx)
        return carry

    lax.fori_loop(0, NIDX, _pos, 0)

    def _drain(g, carry):
        _wait_pair()
        return carry

    lax.fori_loop(0, DP, _drain, 0)

    def _ones(r, carry):
        pltpu.make_async_copy(
            ones_v, out_hbm.at[pos_v.at[r]], ones_sem).start()
        return carry

    lax.fori_loop(0, NIDX, _ones, 0)

    def _odrain(r, carry):
        pltpu.make_async_copy(
            ones_v, out_hbm.at[pos_v.at[0]], ones_sem).wait()
        return carry

    lax.fori_loop(0, NIDX, _odrain, 0)


def kernel(x):
    x = x.reshape(-1).astype(jnp.int32)
    out = _one_hot_sc(x)
    return out.reshape(4096, 26, NUM_COLS)
